# baseline (speedup 1.0000x reference)
import functools

import jax
import jax.numpy as jnp
from jax import lax
from jax.experimental import pallas as pl
from jax.experimental.pallas import tpu as pltpu
from jax.experimental.pallas import tpu_sc as plsc

_NUM_CLASSES = 1000000
_D = 32
_S = 20
_L = 16
_GCHUNK = 128


def _sc_dots(in_embed, out_embed, idx_in, idx_out, idx_noise_flat):
    B = idx_in.shape[0]
    info = plsc.get_sparse_core_info()
    NC, NS = info.num_cores, info.num_subcores
    NW = NC * NS
    CHUNK = B // NW
    SUB = 32
    NSUB = CHUNK // SUB
    NROWS = SUB * _S
    mesh = plsc.VectorSubcoreMesh(core_axis_name="c", subcore_axis_name="s")

    @functools.partial(
        pl.kernel,
        out_type=jax.ShapeDtypeStruct((B, _D), jnp.float32),
        mesh=mesh,
        compiler_params=pltpu.CompilerParams(needs_layout_passes=False,
                                             use_tc_tiling_on_sc=False),
        scratch_types=[
            pltpu.VMEM((CHUNK,), jnp.int32),
            pltpu.VMEM((CHUNK,), jnp.int32),
            pltpu.VMEM((CHUNK * _S,), jnp.int32),
            pltpu.VMEM((CHUNK, _D), jnp.float32),
            pltpu.VMEM((CHUNK, _D), jnp.float32),
            pltpu.VMEM((3, NROWS, _D), jnp.float32),
            pltpu.VMEM((CHUNK, _D), jnp.float32),
            pltpu.SemaphoreType.DMA,
            pltpu.SemaphoreType.DMA,
            pltpu.SemaphoreType.DMA,
            pltpu.SemaphoreType.DMA,
        ],
    )
    def body(in_hbm, out_hbm, ii_hbm, io_hbm, inz_hbm, dots_hbm,
             ii_v, io_v, inz_v, inp_v, outr_v, nz_v, dots_v,
             sem_io, sem_n0, sem_n1, sem_n2):
        wid = lax.axis_index("s") * NC + lax.axis_index("c")
        base = wid * CHUNK

        pltpu.sync_copy(ii_hbm.at[pl.ds(base, CHUNK)], ii_v)
        pltpu.sync_copy(io_hbm.at[pl.ds(base, CHUNK)], io_v)
        pltpu.sync_copy(inz_hbm.at[pl.ds(base * _S, CHUNK * _S)], inz_v)

        io_copies = [
            pltpu.async_copy(in_hbm.at[ii_v], inp_v, sem_io),
            pltpu.async_copy(out_hbm.at[io_v], outr_v, sem_io),
        ]

        sems = (sem_n0, sem_n1, sem_n2)
        NBUF = 3

        def launch_noise(sub, buf):
            return [pltpu.async_copy(
                out_hbm.at[inz_v.at[pl.ds(sub * NROWS, NROWS)]],
                nz_v.at[buf], sems[buf])]

        pending = {b: [] for b in range(NBUF)}
        for sub in range(NBUF - 1):
            pending[sub] = launch_noise(sub, sub)
        for c in io_copies:
            c.wait()

        iota16 = lax.iota(jnp.int32, _L)

        def compute_sub(sub, buf):
            for bb in range(SUB // _L):
                r0 = sub * SUB + bb * _L
                riota = r0 + iota16
                niota = (bb * _L + iota16) * _S

                nio = [niota + s for s in range(_S)]

                def dbody(dcol, carry):
                    colv = jnp.full((_L,), dcol, jnp.int32)
                    ic = plsc.load_gather(inp_v, [riota, colv])
                    oc = plsc.load_gather(outr_v, [riota, colv])
                    ncs = [plsc.load_gather(nz_v.at[buf], [nio[s], colv])
                           for s in range(_S)]
                    accs = [carry[0] + ic * oc]
                    for s in range(_S):
                        accs.append(carry[s + 1] + ic * ncs[s])
                    return tuple(accs)

                zero = jnp.zeros((_L,), jnp.float32)
                accs = lax.fori_loop(0, _D, dbody, (zero,) * (_S + 1))
                for s in range(_S + 1):
                    plsc.store_scatter(dots_v, [riota,
                                                jnp.full((_L,), s, jnp.int32)],
                                       accs[s])

        for sub in range(NSUB):
            buf = sub % NBUF
            nxt = sub + NBUF - 1
            if nxt < NSUB:
                pending[nxt % NBUF] = launch_noise(nxt, nxt % NBUF)
            for c in pending[buf]:
                c.wait()
            compute_sub(sub, buf)

        pltpu.sync_copy(dots_v, dots_hbm.at[pl.ds(base, CHUNK)])

    return body(in_embed, out_embed, idx_in, idx_out, idx_noise_flat)


def _tc_loss(dots):
    B = dots.shape[0]
    BLK = 2048

    def body(d_ref, o_ref):
        x = d_ref[...]
        col = lax.broadcasted_iota(jnp.int32, (BLK, _D), 1)
        y = jnp.where(col == 0, -x, x)
        sp = jnp.maximum(y, 0.0) + jnp.log(1.0 + jnp.exp(-jnp.abs(y)))
        sp = jnp.where(col <= _S, sp, 0.0)
        o_ref[...] = jnp.sum(sp, axis=1)

    return pl.pallas_call(
        body,
        grid=(B // BLK,),
        in_specs=[pl.BlockSpec((BLK, _D), lambda i: (i, 0))],
        out_specs=pl.BlockSpec((BLK,), lambda i: (i,)),
        out_shape=jax.ShapeDtypeStruct((B,), jnp.float32),
    )(dots)


def kernel(in_embed_weight, out_embed_weight, input_labes, out_labels,
           num_sampled):
    B = input_labes.shape[0]
    noise_key = jax.random.key(1234)
    noise_idx = jax.random.randint(noise_key, (B, _S), 0,
                                   _NUM_CLASSES - 1).astype(jnp.int32)
    noise_idx = noise_idx.reshape(-1)
    dots = _sc_dots(in_embed_weight, out_embed_weight,
                    input_labes.astype(jnp.int32),
                    out_labels.astype(jnp.int32),
                    noise_idx)
    return _tc_loss(dots)


# speedup vs baseline: 1.1984x; 1.1984x over previous
import functools

import jax
import jax.numpy as jnp
from jax import lax
from jax.experimental import pallas as pl
from jax.experimental.pallas import tpu as pltpu
from jax.experimental.pallas import tpu_sc as plsc

_NUM_CLASSES = 1000000
_D = 32
_S = 20
_L = 16
_GCHUNK = 128


def _sc_dots(in_embed, out_embed, idx_in, idx_out, idx_noise_flat):
    B = idx_in.shape[0]
    info = plsc.get_sparse_core_info()
    NC, NS = info.num_cores, info.num_subcores
    NW = NC * NS
    CHUNK = B // NW
    SUB = 32
    NSUB = CHUNK // SUB
    NROWS = SUB * _S
    mesh = plsc.VectorSubcoreMesh(core_axis_name="c", subcore_axis_name="s")

    @functools.partial(
        pl.kernel,
        out_type=jax.ShapeDtypeStruct((B, _D), jnp.float32),
        mesh=mesh,
        compiler_params=pltpu.CompilerParams(needs_layout_passes=False,
                                             use_tc_tiling_on_sc=False),
        scratch_types=[
            pltpu.VMEM((CHUNK,), jnp.int32),
            pltpu.VMEM((CHUNK,), jnp.int32),
            pltpu.VMEM((CHUNK * _S,), jnp.int32),
            pltpu.VMEM((CHUNK, _D), jnp.float32),
            pltpu.VMEM((CHUNK, _D), jnp.float32),
            pltpu.VMEM((3, NROWS, _D), jnp.float32),
            pltpu.VMEM((CHUNK, _D), jnp.float32),
            pltpu.SemaphoreType.DMA,
            pltpu.SemaphoreType.DMA,
            pltpu.SemaphoreType.DMA,
            pltpu.SemaphoreType.DMA,
        ],
    )
    def body(in_hbm, out_hbm, ii_hbm, io_hbm, inz_hbm, dots_hbm,
             ii_v, io_v, inz_v, inp_v, outr_v, nz_v, dots_v,
             sem_io, sem_n0, sem_n1, sem_n2):
        wid = lax.axis_index("s") * NC + lax.axis_index("c")
        base = wid * CHUNK

        pltpu.sync_copy(ii_hbm.at[pl.ds(base, CHUNK)], ii_v)
        pltpu.sync_copy(io_hbm.at[pl.ds(base, CHUNK)], io_v)
        pltpu.sync_copy(inz_hbm.at[pl.ds(base * _S, CHUNK * _S)], inz_v)

        io_copies = [
            pltpu.async_copy(in_hbm.at[ii_v], inp_v, sem_io),
            pltpu.async_copy(out_hbm.at[io_v], outr_v, sem_io),
        ]

        sems = (sem_n0, sem_n1, sem_n2)
        NBUF = 3

        def launch_noise(sub, buf):
            return [pltpu.async_copy(
                out_hbm.at[inz_v.at[pl.ds(sub * NROWS, NROWS)]],
                nz_v.at[buf], sems[buf])]

        pending = {b: [] for b in range(NBUF)}
        for sub in range(NBUF - 1):
            pending[sub] = launch_noise(sub, sub)
        for c in io_copies:
            c.wait()

        iota16 = lax.iota(jnp.int32, _L)

        def compute_sub(sub, buf):
            for bb in range(SUB // _L):
                r0 = sub * SUB + bb * _L
                riota = r0 + iota16
                niota = (bb * _L + iota16) * _S

                nio = [niota + s for s in range(_S)]

                def dbody(dcol, carry):
                    colv = (iota16 + dcol) & (_D - 1)
                    ic = plsc.load_gather(inp_v, [riota, colv])
                    oc = plsc.load_gather(outr_v, [riota, colv])
                    ncs = [plsc.load_gather(nz_v.at[buf], [nio[s], colv])
                           for s in range(_S)]
                    accs = [carry[0] + ic * oc]
                    for s in range(_S):
                        accs.append(carry[s + 1] + ic * ncs[s])
                    return tuple(accs)

                zero = jnp.zeros((_L,), jnp.float32)
                accs = lax.fori_loop(0, _D, dbody, (zero,) * (_S + 1))
                for s in range(_S + 1):
                    plsc.store_scatter(dots_v, [riota,
                                                jnp.full((_L,), s, jnp.int32)],
                                       accs[s])

        for sub in range(NSUB):
            buf = sub % NBUF
            nxt = sub + NBUF - 1
            if nxt < NSUB:
                pending[nxt % NBUF] = launch_noise(nxt, nxt % NBUF)
            for c in pending[buf]:
                c.wait()
            compute_sub(sub, buf)

        pltpu.sync_copy(dots_v, dots_hbm.at[pl.ds(base, CHUNK)])

    return body(in_embed, out_embed, idx_in, idx_out, idx_noise_flat)


def _tc_loss(dots):
    B = dots.shape[0]
    BLK = 2048

    def body(d_ref, o_ref):
        x = d_ref[...]
        col = lax.broadcasted_iota(jnp.int32, (BLK, _D), 1)
        y = jnp.where(col == 0, -x, x)
        sp = jnp.maximum(y, 0.0) + jnp.log(1.0 + jnp.exp(-jnp.abs(y)))
        sp = jnp.where(col <= _S, sp, 0.0)
        o_ref[...] = jnp.sum(sp, axis=1)

    return pl.pallas_call(
        body,
        grid=(B // BLK,),
        in_specs=[pl.BlockSpec((BLK, _D), lambda i: (i, 0))],
        out_specs=pl.BlockSpec((BLK,), lambda i: (i,)),
        out_shape=jax.ShapeDtypeStruct((B,), jnp.float32),
    )(dots)


def kernel(in_embed_weight, out_embed_weight, input_labes, out_labels,
           num_sampled):
    B = input_labes.shape[0]
    noise_key = jax.random.key(1234)
    noise_idx = jax.random.randint(noise_key, (B, _S), 0,
                                   _NUM_CLASSES - 1).astype(jnp.int32)
    noise_idx = noise_idx.reshape(-1)
    dots = _sc_dots(in_embed_weight, out_embed_weight,
                    input_labes.astype(jnp.int32),
                    out_labels.astype(jnp.int32),
                    noise_idx)
    return _tc_loss(dots)
